# initial kernel scaffold (unmeasured)
import jax
import jax.numpy as jnp
from jax import lax
from jax.experimental import pallas as pl
from jax.experimental.pallas import tpu as pltpu

N_DEV = 4
E = 32
E_LOC = 8
CAP = 51
CAPP = 64
N = 2048
D = 1024


def _moe_ring_allgather(gathered, expert_W):

    def body(g_ref, w_ref, out_ref, comm_ref, send_sems, recv_sems):
        my = lax.axis_index("i")
        left = lax.rem(my + (N_DEV - 1), N_DEV)
        right = lax.rem(my + 1, N_DEV)

        barrier_sem = pltpu.get_barrier_semaphore()
        for nbr in (left, right):
            pl.semaphore_signal(
                barrier_sem, inc=1,
                device_id=(nbr,), device_id_type=pl.DeviceIdType.MESH,
            )
        pl.semaphore_wait(barrier_sem, 2)

        for j in range(E_LOC):
            comm_ref[0, j] = jnp.dot(
                g_ref[j], w_ref[j], preferred_element_type=jnp.float32
            )
        out_ref[pl.ds(my * E_LOC, E_LOC)] = comm_ref[0]

        for h in range(N_DEV - 1):
            send_slot = h % 2
            recv_slot = (h + 1) % 2
            rdma = pltpu.make_async_remote_copy(
                src_ref=comm_ref.at[send_slot],
                dst_ref=comm_ref.at[recv_slot],
                send_sem=send_sems.at[send_slot],
                recv_sem=recv_sems.at[recv_slot],
                device_id=(right,),
                device_id_type=pl.DeviceIdType.MESH,
            )
            rdma.start()
            rdma.wait()
            origin = lax.rem(my + (N_DEV - 1 - h), N_DEV)
            out_ref[pl.ds(origin * E_LOC, E_LOC)] = comm_ref[recv_slot]

    return pl.pallas_call(
        body,
        out_shape=jax.ShapeDtypeStruct((E, CAPP, D), jnp.float32),
        in_specs=[
            pl.BlockSpec(memory_space=pltpu.VMEM),
            pl.BlockSpec(memory_space=pltpu.VMEM),
        ],
        out_specs=pl.BlockSpec(memory_space=pltpu.VMEM),
        scratch_shapes=[
            pltpu.VMEM((2, E_LOC, CAPP, D), jnp.float32),
            pltpu.SemaphoreType.DMA((2,)),
            pltpu.SemaphoreType.DMA((2,)),
        ],
        compiler_params=pltpu.CompilerParams(collective_id=0),
    )(gathered, expert_W)


def kernel(x, router_W, route_idx, expert_W):
    del router_W

    e_of = route_idx[:, 0].astype(jnp.int32)
    onehot = e_of[:, None] == jnp.arange(E, dtype=jnp.int32)
    pos = jnp.cumsum(onehot.astype(jnp.int32), axis=0)
    slot = jnp.sum(jnp.where(onehot, pos - 1, 0), axis=1)
    kept = slot < CAP
    dest = jnp.where(kept, e_of * CAPP + slot, E * CAPP)

    tok_ids = jnp.arange(N, dtype=jnp.int32)
    tok_for_slot = (
        jnp.zeros(E * CAPP + 1, jnp.int32).at[dest].set(tok_ids)[: E * CAPP]
    )
    valid = (
        jnp.zeros(E * CAPP + 1, jnp.float32).at[dest].set(1.0)[: E * CAPP]
    )

    my = lax.axis_index("i")
    base = my * (E_LOC * CAPP)
    my_slots = lax.dynamic_slice(tok_for_slot, (base,), (E_LOC * CAPP,))
    my_valid = lax.dynamic_slice(valid, (base,), (E_LOC * CAPP,))
    gathered = (x[my_slots] * my_valid[:, None]).reshape(E_LOC, CAPP, D)

    all_compact = _moe_ring_allgather(gathered, expert_W)

    flat = jnp.concatenate(
        [all_compact.reshape(E * CAPP, D), jnp.zeros((1, D), jnp.float32)], axis=0
    )
    return flat[dest]


# baseline (device time: 702212 ns/iter reference)
import jax
import jax.numpy as jnp
from jax import lax
from jax.experimental import pallas as pl
from jax.experimental.pallas import tpu as pltpu

N_DEV = 4
E = 32
E_LOC = 8
CAP = 51
CAPP = 64
N = 2048
D = 1024


def _moe_ring_allgather(gathered, expert_W):

    def body(g_ref, w_ref, out_ref, comm_ref, send_sems, recv_sems):
        my = lax.axis_index("i")
        left = lax.rem(my + (N_DEV - 1), N_DEV)
        right = lax.rem(my + 1, N_DEV)

        barrier_sem = pltpu.get_barrier_semaphore()
        for nbr in (left, right):
            pl.semaphore_signal(
                barrier_sem, inc=1,
                device_id=(nbr,), device_id_type=pl.DeviceIdType.MESH,
            )
        pl.semaphore_wait(barrier_sem, 2)

        for j in range(E_LOC):
            comm_ref[0, j] = jnp.dot(
                g_ref[j], w_ref[j], preferred_element_type=jnp.float32
            )
        out_ref[pl.ds(my * E_LOC, E_LOC)] = comm_ref[0]

        for h in range(N_DEV - 1):
            send_slot = h % 2
            recv_slot = (h + 1) % 2
            rdma = pltpu.make_async_remote_copy(
                src_ref=comm_ref.at[send_slot],
                dst_ref=comm_ref.at[recv_slot],
                send_sem=send_sems.at[send_slot],
                recv_sem=recv_sems.at[recv_slot],
                device_id=(right,),
                device_id_type=pl.DeviceIdType.MESH,
            )
            rdma.start()
            rdma.wait()
            origin = lax.rem(my + (N_DEV - 1 - h), N_DEV)
            out_ref[pl.ds(origin * E_LOC, E_LOC)] = comm_ref[recv_slot]

    return pl.pallas_call(
        body,
        out_shape=jax.ShapeDtypeStruct((E, CAPP, D), jnp.float32),
        in_specs=[
            pl.BlockSpec(memory_space=pltpu.VMEM),
            pl.BlockSpec(memory_space=pltpu.VMEM),
        ],
        out_specs=pl.BlockSpec(memory_space=pltpu.VMEM),
        scratch_shapes=[
            pltpu.VMEM((2, E_LOC, CAPP, D), jnp.float32),
            pltpu.SemaphoreType.DMA((2,)),
            pltpu.SemaphoreType.DMA((2,)),
        ],
        compiler_params=pltpu.CompilerParams(
            collective_id=0, vmem_limit_bytes=100 * 1024 * 1024
        ),
    )(gathered, expert_W)


def kernel(x, router_W, route_idx, expert_W):
    del router_W

    e_of = route_idx[:, 0].astype(jnp.int32)
    onehot = e_of[:, None] == jnp.arange(E, dtype=jnp.int32)
    pos = jnp.cumsum(onehot.astype(jnp.int32), axis=0)
    slot = jnp.sum(jnp.where(onehot, pos - 1, 0), axis=1)
    kept = slot < CAP
    dest = jnp.where(kept, e_of * CAPP + slot, E * CAPP)

    tok_ids = jnp.arange(N, dtype=jnp.int32)
    tok_for_slot = (
        jnp.zeros(E * CAPP + 1, jnp.int32).at[dest].set(tok_ids)[: E * CAPP]
    )
    valid = (
        jnp.zeros(E * CAPP + 1, jnp.float32).at[dest].set(1.0)[: E * CAPP]
    )

    my = lax.axis_index("i")
    base = my * (E_LOC * CAPP)
    my_slots = lax.dynamic_slice(tok_for_slot, (base,), (E_LOC * CAPP,))
    my_valid = lax.dynamic_slice(valid, (base,), (E_LOC * CAPP,))
    gathered = (x[my_slots] * my_valid[:, None]).reshape(E_LOC, CAPP, D)

    all_compact = _moe_ring_allgather(gathered, expert_W)

    flat = jnp.concatenate(
        [all_compact.reshape(E * CAPP, D), jnp.zeros((1, D), jnp.float32)], axis=0
    )
    return flat[dest]


# device time: 142126 ns/iter; 4.9408x vs baseline; 4.9408x over previous
import jax
import jax.numpy as jnp
from jax import lax
from jax.experimental import pallas as pl
from jax.experimental.pallas import tpu as pltpu

N_DEV = 4
E = 32
E_LOC = 8
CAP = 51
CAPP = 64
BLK = E_LOC * CAPP
N = 2048
D = 1024


def _fused_moe(x, expert_W, gather_tok, scat_tok):

    def body(x_ref, w_ref, gtok_ref, stok_ref, out_ref,
             gbuf_ref, comm_ref, send_sems, recv_sems):
        my = lax.axis_index("i")
        left = lax.rem(my + (N_DEV - 1), N_DEV)
        right = lax.rem(my + 1, N_DEV)

        barrier_sem = pltpu.get_barrier_semaphore()
        for nbr in (left, right):
            pl.semaphore_signal(
                barrier_sem, inc=1,
                device_id=(nbr,), device_id_type=pl.DeviceIdType.MESH,
            )
        pl.semaphore_wait(barrier_sem, 2)

        out_ref[:, :] = jnp.zeros((N + 8, D), jnp.float32)

        gbase = my * BLK

        def gbody(k, c):
            t = gtok_ref[gbase + k]
            gbuf_ref[pl.ds(k, 1), :] = x_ref[pl.ds(t, 1), :]
            return c

        lax.fori_loop(0, BLK, gbody, 0, unroll=8)

        for j in range(E_LOC):
            comm_ref[0, pl.ds(j * CAPP, CAPP), :] = jnp.dot(
                gbuf_ref[pl.ds(j * CAPP, CAPP), :],
                w_ref[j],
                preferred_element_type=jnp.float32,
            )

        def scatter_block(slot_idx, origin):
            sbase = origin * BLK

            def sbody(k, c):
                t = stok_ref[sbase + k]
                out_ref[pl.ds(t, 1), :] = comm_ref[slot_idx, pl.ds(k, 1), :]
                return c

            lax.fori_loop(0, BLK, sbody, 0, unroll=8)

        scatter_block(0, my)

        for h in range(N_DEV - 1):
            send_slot = h % 2
            recv_slot = (h + 1) % 2
            rdma = pltpu.make_async_remote_copy(
                src_ref=comm_ref.at[send_slot],
                dst_ref=comm_ref.at[recv_slot],
                send_sem=send_sems.at[send_slot],
                recv_sem=recv_sems.at[recv_slot],
                device_id=(right,),
                device_id_type=pl.DeviceIdType.MESH,
            )
            rdma.start()
            rdma.wait()
            origin = lax.rem(my + (N_DEV - 1 - h), N_DEV)
            scatter_block(recv_slot, origin)

    return pl.pallas_call(
        body,
        out_shape=jax.ShapeDtypeStruct((N + 8, D), jnp.float32),
        in_specs=[
            pl.BlockSpec(memory_space=pltpu.VMEM),
            pl.BlockSpec(memory_space=pltpu.VMEM),
            pl.BlockSpec(memory_space=pltpu.SMEM),
            pl.BlockSpec(memory_space=pltpu.SMEM),
        ],
        out_specs=pl.BlockSpec(memory_space=pltpu.VMEM),
        scratch_shapes=[
            pltpu.VMEM((BLK, D), jnp.float32),
            pltpu.VMEM((2, BLK, D), jnp.float32),
            pltpu.SemaphoreType.DMA((2,)),
            pltpu.SemaphoreType.DMA((2,)),
        ],
        compiler_params=pltpu.CompilerParams(
            collective_id=0, vmem_limit_bytes=100 * 1024 * 1024
        ),
    )(x, expert_W, gather_tok, scat_tok)


def kernel(x, router_W, route_idx, expert_W):
    del router_W

    e_of = route_idx[:, 0].astype(jnp.int32)
    onehot = e_of[:, None] == jnp.arange(E, dtype=jnp.int32)
    pos = jnp.cumsum(onehot.astype(jnp.int32), axis=0)
    slot = jnp.sum(jnp.where(onehot, pos - 1, 0), axis=1)
    kept = slot < CAP
    dest = jnp.where(kept, e_of * CAPP + slot, E * CAPP)

    tok_ids = jnp.arange(N, dtype=jnp.int32)
    gather_tok = (
        jnp.zeros(E * CAPP + 1, jnp.int32).at[dest].set(tok_ids)[: E * CAPP]
    )
    scat_tok = (
        jnp.full(E * CAPP + 1, N, jnp.int32).at[dest].set(tok_ids)[: E * CAPP]
    )

    y = _fused_moe(x, expert_W, gather_tok, scat_tok)
    return y[:N]


# device time: 130567 ns/iter; 5.3782x vs baseline; 1.0885x over previous
import jax
import jax.numpy as jnp
from jax import lax
from jax.experimental import pallas as pl
from jax.experimental.pallas import tpu as pltpu

N_DEV = 4
E = 32
E_LOC = 8
CAP = 51
CAPP = 64
BLK = E_LOC * CAPP
N = 2048
D = 1024


def _fused_moe(x, expert_W, slot_tok):

    def body(x_ref, w_ref, stok_ref, out_ref,
             gbuf_ref, comm_ref, send_sems, recv_sems):
        my = lax.axis_index("i")
        left = lax.rem(my + (N_DEV - 1), N_DEV)
        right = lax.rem(my + 1, N_DEV)

        barrier_sem = pltpu.get_barrier_semaphore()
        for nbr in (left, right):
            pl.semaphore_signal(
                barrier_sem, inc=1,
                device_id=(nbr,), device_id_type=pl.DeviceIdType.MESH,
            )
        pl.semaphore_wait(barrier_sem, 2)

        gbase = my * BLK

        def gbody(k, c):
            t = jnp.minimum(stok_ref[gbase + k], N - 1)
            gbuf_ref[pl.ds(k, 1), :] = x_ref[pl.ds(t, 1), :]
            return c

        lax.fori_loop(0, BLK, gbody, 0, unroll=8)

        for j in range(E_LOC):
            comm_ref[0, pl.ds(j * CAPP, CAPP), :] = jnp.dot(
                gbuf_ref[pl.ds(j * CAPP, CAPP), :],
                w_ref[j],
                preferred_element_type=jnp.float32,
            )

        def scatter_block(slot_idx, origin):
            sbase = origin * BLK

            def sbody(k, c):
                t = stok_ref[sbase + k]

                @pl.when(t < N)
                def _():
                    out_ref[pl.ds(t, 1), :] = comm_ref[slot_idx, pl.ds(k, 1), :]

                return c

            lax.fori_loop(0, BLK, sbody, 0, unroll=8)

        for h in range(N_DEV - 1):
            send_slot = h % 2
            recv_slot = (h + 1) % 2
            rdma = pltpu.make_async_remote_copy(
                src_ref=comm_ref.at[send_slot],
                dst_ref=comm_ref.at[recv_slot],
                send_sem=send_sems.at[send_slot],
                recv_sem=recv_sems.at[recv_slot],
                device_id=(right,),
                device_id_type=pl.DeviceIdType.MESH,
            )
            rdma.start()
            if h == 0:
                out_ref[:, :] = jnp.zeros((N, D), jnp.float32)
            scatter_block(send_slot, lax.rem(my + (N_DEV - h), N_DEV))
            rdma.wait()
        scatter_block((N_DEV - 1) % 2, lax.rem(my + 1, N_DEV))

    return pl.pallas_call(
        body,
        out_shape=jax.ShapeDtypeStruct((N, D), jnp.float32),
        in_specs=[
            pl.BlockSpec(memory_space=pltpu.VMEM),
            pl.BlockSpec(memory_space=pltpu.VMEM),
            pl.BlockSpec(memory_space=pltpu.SMEM),
        ],
        out_specs=pl.BlockSpec(memory_space=pltpu.VMEM),
        scratch_shapes=[
            pltpu.VMEM((BLK, D), jnp.float32),
            pltpu.VMEM((2, BLK, D), jnp.float32),
            pltpu.SemaphoreType.DMA((2,)),
            pltpu.SemaphoreType.DMA((2,)),
        ],
        compiler_params=pltpu.CompilerParams(
            collective_id=0, vmem_limit_bytes=100 * 1024 * 1024
        ),
    )(x, expert_W, slot_tok)


def kernel(x, router_W, route_idx, expert_W):
    del router_W

    e_of = route_idx[:, 0].astype(jnp.int32)
    onehot = e_of[:, None] == jnp.arange(E, dtype=jnp.int32)
    pos = jnp.cumsum(onehot.astype(jnp.int32), axis=0)
    slot = jnp.sum(jnp.where(onehot, pos - 1, 0), axis=1)
    kept = slot < CAP
    dest = jnp.where(kept, e_of * CAPP + slot, E * CAPP)

    tok_ids = jnp.arange(N, dtype=jnp.int32)
    slot_tok = (
        jnp.full(E * CAPP + 1, N, jnp.int32).at[dest].set(tok_ids)[: E * CAPP]
    )

    return _fused_moe(x, expert_W, slot_tok)


# device time: 89773 ns/iter; 7.8221x vs baseline; 1.4544x over previous
import jax
import jax.numpy as jnp
from jax import lax
from jax.experimental import pallas as pl
from jax.experimental.pallas import tpu as pltpu

N_DEV = 4
E = 32
E_LOC = 8
CAP = 51
CAPP = 56
BLK = E_LOC * CAPP
HLF = BLK // 2
N = 2048
D = 1024


def _fused_moe(x, expert_W, slot_tok):
    def body(x_ref, w_ref, stok_ref, out_ref,
             gbuf_ref, comm_ref, send_sems, recv_sems):
        my = lax.axis_index("i")
        left = lax.rem(my + (N_DEV - 1), N_DEV)
        right = lax.rem(my + 1, N_DEV)

        barrier_sem = pltpu.get_barrier_semaphore()
        for nbr in (left, right):
            pl.semaphore_signal(
                barrier_sem, inc=1,
                device_id=(nbr,), device_id_type=pl.DeviceIdType.MESH,
            )
        pl.semaphore_wait(barrier_sem, 2)

        gbase = my * BLK

        def gbody(k, c):
            t = jnp.minimum(stok_ref[gbase + k], N - 1)
            gbuf_ref[pl.ds(k, 1), :] = x_ref[pl.ds(t, 1), :]
            return c

        lax.fori_loop(0, BLK, gbody, 0, unroll=8)

        for j in range(E_LOC):
            comm_ref[0, pl.ds(j * CAPP, CAPP), :] = jnp.dot(
                gbuf_ref[pl.ds(j * CAPP, CAPP), :],
                w_ref[j],
                preferred_element_type=jnp.float32,
            )

        def scatter_block(slot_idx, origin):
            sbase = origin * BLK

            def sbody(k, c):
                t = stok_ref[sbase + k]

                @pl.when(t < N)
                def _():
                    out_ref[pl.ds(t, 1), :] = comm_ref[slot_idx, pl.ds(k, 1), :]

                return c

            lax.fori_loop(0, BLK, sbody, 0, unroll=8)

        r1 = pltpu.make_async_remote_copy(
            src_ref=comm_ref.at[0], dst_ref=comm_ref.at[1],
            send_sem=send_sems.at[0], recv_sem=recv_sems.at[0],
            device_id=(right,), device_id_type=pl.DeviceIdType.MESH,
        )
        l1 = pltpu.make_async_remote_copy(
            src_ref=comm_ref.at[0], dst_ref=comm_ref.at[2],
            send_sem=send_sems.at[1], recv_sem=recv_sems.at[1],
            device_id=(left,), device_id_type=pl.DeviceIdType.MESH,
        )
        r1.start()
        l1.start()

        out_ref[:, :] = jnp.zeros((N, D), jnp.float32)
        scatter_block(0, my)

        r1.wait_recv()
        r2 = pltpu.make_async_remote_copy(
            src_ref=comm_ref.at[1, pl.ds(0, HLF), :],
            dst_ref=comm_ref.at[3, pl.ds(0, HLF), :],
            send_sem=send_sems.at[2], recv_sem=recv_sems.at[2],
            device_id=(right,), device_id_type=pl.DeviceIdType.MESH,
        )
        r2.start()
        l1.wait_recv()
        l2 = pltpu.make_async_remote_copy(
            src_ref=comm_ref.at[2, pl.ds(HLF, HLF), :],
            dst_ref=comm_ref.at[3, pl.ds(HLF, HLF), :],
            send_sem=send_sems.at[3], recv_sem=recv_sems.at[3],
            device_id=(left,), device_id_type=pl.DeviceIdType.MESH,
        )
        l2.start()

        scatter_block(1, left)
        scatter_block(2, right)

        r2.wait_recv()
        l2.wait_recv()
        scatter_block(3, lax.rem(my + 2, N_DEV))

        r1.wait_send()
        l1.wait_send()
        r2.wait_send()
        l2.wait_send()

    return pl.pallas_call(
        body,
        out_shape=jax.ShapeDtypeStruct((N, D), jnp.float32),
        in_specs=[
            pl.BlockSpec(memory_space=pltpu.VMEM),
            pl.BlockSpec(memory_space=pltpu.VMEM),
            pl.BlockSpec(memory_space=pltpu.SMEM),
        ],
        out_specs=pl.BlockSpec(memory_space=pltpu.VMEM),
        scratch_shapes=[
            pltpu.VMEM((BLK, D), jnp.float32),
            pltpu.VMEM((N_DEV, BLK, D), jnp.float32),
            pltpu.SemaphoreType.DMA((4,)),
            pltpu.SemaphoreType.DMA((4,)),
        ],
        compiler_params=pltpu.CompilerParams(
            collective_id=0, vmem_limit_bytes=110 * 1024 * 1024
        ),
    )(x, expert_W, slot_tok)


def kernel(x, router_W, route_idx, expert_W):
    del router_W

    e_of = route_idx[:, 0].astype(jnp.int32)
    onehot = e_of[:, None] == jnp.arange(E, dtype=jnp.int32)
    pos = jnp.cumsum(onehot.astype(jnp.int32), axis=0)
    slot = jnp.sum(jnp.where(onehot, pos - 1, 0), axis=1)
    kept = slot < CAP
    dest = jnp.where(kept, e_of * CAPP + slot, E * CAPP)

    tok_ids = jnp.arange(N, dtype=jnp.int32)
    slot_tok = (
        jnp.full(E * CAPP + 1, N, jnp.int32).at[dest].set(tok_ids)[: E * CAPP]
    )

    return _fused_moe(x, expert_W, slot_tok)


# device time: 78983 ns/iter; 8.8907x vs baseline; 1.1366x over previous
import jax
import jax.numpy as jnp
from jax import lax
from jax.experimental import pallas as pl
from jax.experimental.pallas import tpu as pltpu

N_DEV = 4
E = 32
E_LOC = 8
CAP = 51
CAPP = 56
BLK = E_LOC * CAPP
HLF = BLK // 2
N = 2048
D = 1024


def _fused_moe(x, expert_W, slot_tok):
    def body(x_ref, w_ref, stok_ref, out_ref,
             gbuf_ref, ybuf_ref, comm_ref, send_sems, recv_sems):
        my = lax.axis_index("i")
        left = lax.rem(my + (N_DEV - 1), N_DEV)
        right = lax.rem(my + 1, N_DEV)

        barrier_sem = pltpu.get_barrier_semaphore()
        for nbr in (left, right):
            pl.semaphore_signal(
                barrier_sem, inc=1,
                device_id=(nbr,), device_id_type=pl.DeviceIdType.MESH,
            )
        pl.semaphore_wait(barrier_sem, 2)

        gbase = my * BLK

        def gbody(k, c):
            t = jnp.minimum(stok_ref[gbase + k], N - 1)
            gbuf_ref[pl.ds(k, 1), :] = x_ref[pl.ds(t, 1), :]
            return c

        lax.fori_loop(0, BLK, gbody, 0, unroll=8)

        for j in range(E_LOC):
            ybuf_ref[pl.ds(j * CAPP, CAPP), :] = jnp.dot(
                gbuf_ref[pl.ds(j * CAPP, CAPP), :],
                w_ref[j],
                preferred_element_type=jnp.float32,
            )
        comm_ref[0, :, :] = ybuf_ref[:, :].astype(jnp.bfloat16)

        def scatter_block(origin):
            sbase = origin * BLK

            def sbody(k, c):
                t = stok_ref[sbase + k]

                @pl.when(t < N)
                def _():
                    out_ref[pl.ds(t, 1), :] = ybuf_ref[pl.ds(k, 1), :]

                return c

            lax.fori_loop(0, BLK, sbody, 0, unroll=8)

        r1 = pltpu.make_async_remote_copy(
            src_ref=comm_ref.at[0], dst_ref=comm_ref.at[1],
            send_sem=send_sems.at[0], recv_sem=recv_sems.at[0],
            device_id=(right,), device_id_type=pl.DeviceIdType.MESH,
        )
        l1 = pltpu.make_async_remote_copy(
            src_ref=comm_ref.at[0], dst_ref=comm_ref.at[2],
            send_sem=send_sems.at[1], recv_sem=recv_sems.at[1],
            device_id=(left,), device_id_type=pl.DeviceIdType.MESH,
        )
        r1.start()
        l1.start()

        out_ref[:, :] = jnp.zeros((N, D), jnp.float32)
        scatter_block(my)

        r1.wait_recv()
        r2 = pltpu.make_async_remote_copy(
            src_ref=comm_ref.at[1, pl.ds(0, HLF), :],
            dst_ref=comm_ref.at[3, pl.ds(0, HLF), :],
            send_sem=send_sems.at[2], recv_sem=recv_sems.at[2],
            device_id=(right,), device_id_type=pl.DeviceIdType.MESH,
        )
        r2.start()
        l1.wait_recv()
        l2 = pltpu.make_async_remote_copy(
            src_ref=comm_ref.at[2, pl.ds(HLF, HLF), :],
            dst_ref=comm_ref.at[3, pl.ds(HLF, HLF), :],
            send_sem=send_sems.at[3], recv_sem=recv_sems.at[3],
            device_id=(left,), device_id_type=pl.DeviceIdType.MESH,
        )
        l2.start()

        ybuf_ref[:, :] = comm_ref[1, :, :].astype(jnp.float32)
        scatter_block(left)
        ybuf_ref[:, :] = comm_ref[2, :, :].astype(jnp.float32)
        scatter_block(right)

        r2.wait_recv()
        l2.wait_recv()
        ybuf_ref[:, :] = comm_ref[3, :, :].astype(jnp.float32)
        scatter_block(lax.rem(my + 2, N_DEV))

        r1.wait_send()
        l1.wait_send()
        r2.wait_send()
        l2.wait_send()

    return pl.pallas_call(
        body,
        out_shape=jax.ShapeDtypeStruct((N, D), jnp.float32),
        in_specs=[
            pl.BlockSpec(memory_space=pltpu.VMEM),
            pl.BlockSpec(memory_space=pltpu.VMEM),
            pl.BlockSpec(memory_space=pltpu.SMEM),
        ],
        out_specs=pl.BlockSpec(memory_space=pltpu.VMEM),
        scratch_shapes=[
            pltpu.VMEM((BLK, D), jnp.float32),
            pltpu.VMEM((BLK, D), jnp.float32),
            pltpu.VMEM((N_DEV, BLK, D), jnp.bfloat16),
            pltpu.SemaphoreType.DMA((4,)),
            pltpu.SemaphoreType.DMA((4,)),
        ],
        compiler_params=pltpu.CompilerParams(
            collective_id=0, vmem_limit_bytes=110 * 1024 * 1024
        ),
    )(x, expert_W, slot_tok)


def kernel(x, router_W, route_idx, expert_W):
    del router_W

    e_of = route_idx[:, 0].astype(jnp.int32)
    onehot = e_of[:, None] == jnp.arange(E, dtype=jnp.int32)
    pos = jnp.cumsum(onehot.astype(jnp.int32), axis=0)
    slot = jnp.sum(jnp.where(onehot, pos - 1, 0), axis=1)
    kept = slot < CAP
    dest = jnp.where(kept, e_of * CAPP + slot, E * CAPP)

    tok_ids = jnp.arange(N, dtype=jnp.int32)
    slot_tok = (
        jnp.full(E * CAPP + 1, N, jnp.int32).at[dest].set(tok_ids)[: E * CAPP]
    )

    return _fused_moe(x, expert_W, slot_tok)
